# baseline (speedup 1.0000x reference)
import jax
import jax.numpy as jnp
from jax import lax
from jax.experimental import pallas as pl

N = 8192
E = 8
D = 512
DH = D // 2
ROW = E + E * D

TB = 1024
GRID = N // TB


def _fused_body(pT_ref, uT_ref, eps_ref, out_ref):
    u = uT_ref[...]
    uc = jnp.clip(u, 1e-6, 1.0 - 1e-6)
    g = -jnp.log(-jnp.log(uc))
    s = pT_ref[0:E, :] + g
    m = jnp.max(s, axis=0, keepdims=True)
    idx = lax.broadcasted_iota(jnp.int32, (E, TB), 0)
    sel = jnp.min(jnp.where(s == m, idx, E), axis=0, keepdims=True)

    mean = pT_ref[E:E + DH, :]
    lsd = pT_ref[E + DH:E + D, :]
    for e in range(1, E):
        msk = sel == e
        mean = jnp.where(msk, pT_ref[E + e * D:E + e * D + DH, :], mean)
        lsd = jnp.where(msk, pT_ref[E + e * D + DH:E + (e + 1) * D, :], lsd)
    mean_t = jnp.transpose(mean, (1, 0))
    lsd_t = jnp.transpose(lsd, (1, 0))
    out_ref[...] = mean_t + eps_ref[...] * jnp.exp(lsd_t)


def kernel(params, u, eps):
    pT = params.T
    uT = u.T
    return pl.pallas_call(
        _fused_body,
        grid=(GRID,),
        in_specs=[
            pl.BlockSpec((ROW, TB), lambda b: (0, b)),
            pl.BlockSpec((E, TB), lambda b: (0, b)),
            pl.BlockSpec((TB, DH), lambda b: (b, 0)),
        ],
        out_specs=pl.BlockSpec((TB, DH), lambda b: (b, 0)),
        out_shape=jax.ShapeDtypeStruct((N, DH), jnp.float32),
    )(pT, uT, eps)


# speedup vs baseline: 1.0844x; 1.0844x over previous
import jax
import jax.numpy as jnp
from jax import lax
from jax.experimental import pallas as pl

N = 8192
E = 8
D = 512
DH = D // 2
ROW = E + E * D

TB = 512
GRID = N // TB


def _fused_body(pT_ref, uT_ref, eps_ref, out_ref):
    u = uT_ref[...]
    uc = jnp.clip(u, 1e-6, 1.0 - 1e-6)
    g = -jnp.log(-jnp.log(uc))
    s = pT_ref[0:E, :] + g
    m = jnp.max(s, axis=0, keepdims=True)
    idx = lax.broadcasted_iota(jnp.int32, (E, TB), 0)
    sel = jnp.min(jnp.where(s == m, idx, E), axis=0, keepdims=True)

    mean = pT_ref[E:E + DH, :]
    lsd = pT_ref[E + DH:E + D, :]
    for e in range(1, E):
        msk = sel == e
        mean = jnp.where(msk, pT_ref[E + e * D:E + e * D + DH, :], mean)
        lsd = jnp.where(msk, pT_ref[E + e * D + DH:E + (e + 1) * D, :], lsd)
    mean_t = jnp.transpose(mean, (1, 0))
    lsd_t = jnp.transpose(lsd, (1, 0))
    out_ref[...] = mean_t + eps_ref[...] * jnp.exp(lsd_t)


def kernel(params, u, eps):
    pT = params.T
    uT = u.T
    return pl.pallas_call(
        _fused_body,
        grid=(GRID,),
        in_specs=[
            pl.BlockSpec((ROW, TB), lambda b: (0, b)),
            pl.BlockSpec((E, TB), lambda b: (0, b)),
            pl.BlockSpec((TB, DH), lambda b: (b, 0)),
        ],
        out_specs=pl.BlockSpec((TB, DH), lambda b: (b, 0)),
        out_shape=jax.ShapeDtypeStruct((N, DH), jnp.float32),
    )(pT, uT, eps)
